# baseline (speedup 1.0000x reference)
import functools

import jax
import jax.numpy as jnp
from jax.experimental import pallas as pl
from jax.experimental.pallas import tpu as pltpu

B, S, D = 1, 2048, 768
H, KVH, HD = 12, 4, 64
E, TOPK, DFF = 8, 2, 2048
THETA = 1e6
EPS = 1e-5

BS = 256
NSB = S // BS
BM = 128
NMB = S // BM
NEG = -1e30


def _dotT(a, b):
    return jax.lax.dot_general(a, b, (((1,), (1,)), ((), ())),
                               preferred_element_type=jnp.float32)


def _ln1_body(x_ref, w_ref, o_ref):
    x = x_ref[...]
    v = jnp.mean(x * x, axis=-1, keepdims=True)
    xn = x * jax.lax.rsqrt(v + EPS) * w_ref[...]
    o_ref[...] = xn.astype(jnp.bfloat16)


def _kv_body(x_ref, kw_ref, vw_ref, cos_ref, sin_ref, k_ref, v_ref):
    x = x_ref[...]
    kw = kw_ref[0].astype(jnp.bfloat16)
    vw = vw_ref[0].astype(jnp.bfloat16)
    k = _dotT(x, kw)
    k1 = k[:, : HD // 2]
    k2 = k[:, HD // 2:]
    rot = jnp.concatenate([-k2, k1], axis=1)
    kr = k * cos_ref[...] + rot * sin_ref[...]
    k_ref[0] = kr.astype(jnp.bfloat16)
    v_ref[0] = _dotT(x, vw).astype(jnp.bfloat16)


def _attn_body(x_ref, qw_ref, cos_ref, sin_ref, k_ref, v_ref, o_ref):
    qb = pl.program_id(1)
    x = x_ref[...]
    qw = qw_ref[0].astype(jnp.bfloat16)
    q = _dotT(x, qw)
    q1 = q[:, : HD // 2]
    q2 = q[:, HD // 2:]
    rot = jnp.concatenate([-q2, q1], axis=1)
    qr = (q * cos_ref[...] + rot * sin_ref[...]) * (1.0 / 8.0)
    s = _dotT(qr.astype(jnp.bfloat16), k_ref[0])
    row = qb * BS + jax.lax.broadcasted_iota(jnp.int32, (BS, S), 0)
    col = jax.lax.broadcasted_iota(jnp.int32, (BS, S), 1)
    s = jnp.where(col <= row, s, NEG)
    m = jnp.max(s, axis=1, keepdims=True)
    p = jnp.exp(s - m)
    p = p / jnp.sum(p, axis=1, keepdims=True)
    o_ref[0] = jnp.dot(p.astype(jnp.bfloat16), v_ref[0],
                       preferred_element_type=jnp.float32)


def _post_body(o_ref, ow_ref, hid_ref, w2n_ref, gw_ref,
               h2_ref, xn_ref, coef_ref):
    a = _dotT(o_ref[...].astype(jnp.bfloat16), ow_ref[...].astype(jnp.bfloat16))
    h2 = hid_ref[...] + a
    h2_ref[...] = h2
    v = jnp.mean(h2 * h2, axis=-1, keepdims=True)
    xn = h2 * jax.lax.rsqrt(v + EPS) * w2n_ref[...]
    xn_ref[...] = xn.astype(jnp.bfloat16)
    logits = _dotT(xn, gw_ref[...])
    ii = jax.lax.broadcasted_iota(jnp.int32, (BS, E), 1)
    m1 = jnp.max(logits, axis=1, keepdims=True)
    e1 = jnp.min(jnp.where(logits == m1, ii, E), axis=1, keepdims=True)
    masked = jnp.where(ii == e1, NEG, logits)
    m2 = jnp.max(masked, axis=1, keepdims=True)
    e2 = jnp.min(jnp.where(masked == m2, ii, E), axis=1, keepdims=True)
    w1v = 1.0 / (1.0 + jnp.exp(m2 - m1))
    w2v = 1.0 - w1v
    coef_ref[...] = jnp.where(ii == e1, w1v, 0.0) + jnp.where(ii == e2, w2v, 0.0)


def _moe_body(xn_ref, h2_ref, coef_ref, w1_ref, w2_ref, w3_ref, out_ref):
    e = pl.program_id(0)
    b = pl.program_id(1)
    rows = pl.ds(b * BM, BM)
    x = xn_ref[...]
    a = _dotT(x, w1_ref[0])
    u = _dotT(x, w3_ref[0])
    hh = (a * jax.nn.sigmoid(a) * u).astype(jnp.bfloat16)
    y = _dotT(hh, w2_ref[0])
    c = coef_ref[...]
    ii = jax.lax.broadcasted_iota(jnp.int32, (BM, E), 1)
    ce = jnp.sum(jnp.where(ii == e, c, 0.0), axis=1, keepdims=True)
    contrib = y * ce

    @pl.when(e == 0)
    def _():
        out_ref[rows, :] = h2_ref[...] + contrib

    @pl.when(e > 0)
    def _():
        out_ref[rows, :] = out_ref[rows, :] + contrib


def _rope_tables():
    pos = jnp.arange(S, dtype=jnp.float32)
    inv_freq = 1.0 / (THETA ** (jnp.arange(0, HD, 2, dtype=jnp.float32) / HD))
    freqs = pos[:, None] * inv_freq[None, :]
    emb = jnp.concatenate([freqs, freqs], axis=-1)
    return jnp.cos(emb), jnp.sin(emb)


@functools.partial(jax.jit, static_argnames=("interpret",))
def kernel(hidden_states, ln1_w, ln2_w, q_w, k_w, v_w, o_w, gate_w,
           w1, w2, w3, interpret=False):
    x2d = hidden_states.reshape(S, D)
    cos, sin = _rope_tables()

    xn1 = pl.pallas_call(
        _ln1_body,
        grid=(NSB,),
        in_specs=[pl.BlockSpec((BS, D), lambda b: (b, 0)),
                  pl.BlockSpec((1, D), lambda b: (0, 0))],
        out_specs=pl.BlockSpec((BS, D), lambda b: (b, 0)),
        out_shape=jax.ShapeDtypeStruct((S, D), jnp.bfloat16),
        interpret=interpret,
    )(x2d, ln1_w.reshape(1, D))

    kw3 = k_w.reshape(KVH, HD, D)
    vw3 = v_w.reshape(KVH, HD, D)
    k_all, v_all = pl.pallas_call(
        _kv_body,
        grid=(KVH, NSB),
        in_specs=[pl.BlockSpec((BS, D), lambda h, b: (b, 0)),
                  pl.BlockSpec((1, HD, D), lambda h, b: (h, 0, 0)),
                  pl.BlockSpec((1, HD, D), lambda h, b: (h, 0, 0)),
                  pl.BlockSpec((BS, HD), lambda h, b: (b, 0)),
                  pl.BlockSpec((BS, HD), lambda h, b: (b, 0))],
        out_specs=[pl.BlockSpec((1, BS, HD), lambda h, b: (h, b, 0)),
                   pl.BlockSpec((1, BS, HD), lambda h, b: (h, b, 0))],
        out_shape=[jax.ShapeDtypeStruct((KVH, S, HD), jnp.bfloat16),
                   jax.ShapeDtypeStruct((KVH, S, HD), jnp.bfloat16)],
        interpret=interpret,
    )(xn1, kw3, vw3, cos, sin)

    qw3 = q_w.reshape(H, HD, D)
    o_attn = pl.pallas_call(
        _attn_body,
        grid=(H, NSB),
        in_specs=[pl.BlockSpec((BS, D), lambda h, b: (b, 0)),
                  pl.BlockSpec((1, HD, D), lambda h, b: (h, 0, 0)),
                  pl.BlockSpec((BS, HD), lambda h, b: (b, 0)),
                  pl.BlockSpec((BS, HD), lambda h, b: (b, 0)),
                  pl.BlockSpec((1, S, HD), lambda h, b: (h // (H // KVH), 0, 0)),
                  pl.BlockSpec((1, S, HD), lambda h, b: (h // (H // KVH), 0, 0))],
        out_specs=pl.BlockSpec((1, BS, HD), lambda h, b: (h, b, 0)),
        out_shape=jax.ShapeDtypeStruct((H, S, HD), jnp.float32),
        interpret=interpret,
    )(xn1, qw3, cos, sin, k_all, v_all)
    o_attn = o_attn.transpose(1, 0, 2).reshape(S, H * HD)

    h2, xn2, coef = pl.pallas_call(
        _post_body,
        grid=(NSB,),
        in_specs=[pl.BlockSpec((BS, H * HD), lambda b: (b, 0)),
                  pl.BlockSpec((D, H * HD), lambda b: (0, 0)),
                  pl.BlockSpec((BS, D), lambda b: (b, 0)),
                  pl.BlockSpec((1, D), lambda b: (0, 0)),
                  pl.BlockSpec((E, D), lambda b: (0, 0))],
        out_specs=[pl.BlockSpec((BS, D), lambda b: (b, 0)),
                   pl.BlockSpec((BS, D), lambda b: (b, 0)),
                   pl.BlockSpec((BS, E), lambda b: (b, 0))],
        out_shape=[jax.ShapeDtypeStruct((S, D), jnp.float32),
                   jax.ShapeDtypeStruct((S, D), jnp.bfloat16),
                   jax.ShapeDtypeStruct((S, E), jnp.float32)],
        interpret=interpret,
    )(o_attn, o_w, x2d, ln2_w.reshape(1, D), gate_w)

    out = pl.pallas_call(
        _moe_body,
        grid=(E, NMB),
        in_specs=[pl.BlockSpec((BM, D), lambda e, b: (b, 0)),
                  pl.BlockSpec((BM, D), lambda e, b: (b, 0)),
                  pl.BlockSpec((BM, E), lambda e, b: (b, 0)),
                  pl.BlockSpec((1, DFF, D), lambda e, b: (e, 0, 0)),
                  pl.BlockSpec((1, D, DFF), lambda e, b: (e, 0, 0)),
                  pl.BlockSpec((1, DFF, D), lambda e, b: (e, 0, 0))],
        out_specs=pl.BlockSpec((S, D), lambda e, b: (0, 0)),
        out_shape=jax.ShapeDtypeStruct((S, D), jnp.float32),
        interpret=interpret,
    )(xn2, h2, coef,
      w1.astype(jnp.bfloat16), w2.astype(jnp.bfloat16),
      w3.astype(jnp.bfloat16))

    return out.reshape(B, S, D)
